# baseline (speedup 1.0000x reference)
import functools

import jax
import jax.numpy as jnp
from jax import lax
from jax.experimental import pallas as pl
from jax.experimental.pallas import tpu as pltpu
from jax.experimental.pallas import tpu_sc as plsc

_NC = 2
_NS = 16
_L = 16
_NW = _NC * _NS

_CHUNK_ROWS = 16


def _sc_add(x, emb):
    B, N, D = x.shape
    rows_w = N // _NW
    words = _CHUNK_ROWS * D
    n_chunks = rows_w // _CHUNK_ROWS

    xf = x.reshape(B, N * D)
    ef = emb.reshape(N * D)

    mesh = plsc.VectorSubcoreMesh(core_axis_name="c", subcore_axis_name="s")

    @functools.partial(
        pl.kernel,
        out_type=jax.ShapeDtypeStruct((B, N * D), jnp.float32),
        mesh=mesh,
        scratch_types=[
            pltpu.VMEM((words,), jnp.float32),
            pltpu.VMEM((words,), jnp.float32),
        ],
    )
    def k(x_hbm, emb_hbm, out_hbm, x_v, emb_v):
        wid = lax.axis_index("s") * _NC + lax.axis_index("c")
        base = wid * (rows_w * D)

        def chunk_body(c, carry):
            off = base + c * words
            pltpu.sync_copy(emb_hbm.at[pl.ds(off, words)], emb_v)

            def batch_body(b, carry):
                pltpu.sync_copy(x_hbm.at[b, pl.ds(off, words)], x_v)

                def add_body(i, carry):
                    sl = pl.ds(i * _L, _L)
                    x_v[sl] = x_v[sl] + emb_v[sl]
                    return carry

                lax.fori_loop(0, words // _L, add_body, None)
                pltpu.sync_copy(x_v, out_hbm.at[b, pl.ds(off, words)])
                return carry

            lax.fori_loop(0, B, batch_body, None)
            return carry

        lax.fori_loop(0, n_chunks, chunk_body, None)

    return k(xf, ef).reshape(B, N, D)


def kernel(x, emb):
    B, N, D = x.shape
    return _sc_add(x, emb[:N])


# speedup vs baseline: 1.6976x; 1.6976x over previous
import functools

import jax
import jax.numpy as jnp
from jax import lax
from jax.experimental import pallas as pl
from jax.experimental.pallas import tpu as pltpu
from jax.experimental.pallas import tpu_sc as plsc

_NC = 2
_NS = 16
_L = 16
_NW = _NC * _NS

_CHUNK_ROWS = 16

_BLK_N = 2048


def _tc_add(x, emb, batches):
    B, N, D = x.shape
    nb = N // _BLK_N
    return pl.pallas_call(
        _tc_body,
        grid=(nb, batches),
        in_specs=[
            pl.BlockSpec((1, _BLK_N, D), lambda i, b: (b, i, 0)),
            pl.BlockSpec((_BLK_N, D), lambda i, b: (i, 0)),
        ],
        out_specs=pl.BlockSpec((1, _BLK_N, D), lambda i, b: (b, i, 0)),
        out_shape=jax.ShapeDtypeStruct((batches, N, D), x.dtype),
        compiler_params=pltpu.CompilerParams(
            dimension_semantics=("parallel", "parallel"),
        ),
    )(x, emb)


def _tc_body(x_ref, emb_ref, o_ref):
    o_ref[...] = x_ref[...] + emb_ref[...]


def _sc_add_one_batch(x, emb, b_idx):
    B, N, D = x.shape
    rows_w = N // _NW
    words = _CHUNK_ROWS * D
    n_chunks = rows_w // _CHUNK_ROWS

    xf = x.reshape(B, N * D)
    ef = emb.reshape(N * D)

    mesh = plsc.VectorSubcoreMesh(core_axis_name="c", subcore_axis_name="s")

    @functools.partial(
        pl.kernel,
        out_type=jax.ShapeDtypeStruct((N * D,), jnp.float32),
        mesh=mesh,
        scratch_types=[
            pltpu.VMEM((words,), jnp.float32),
            pltpu.VMEM((words,), jnp.float32),
        ],
    )
    def k(x_hbm, emb_hbm, out_hbm, x_v, emb_v):
        wid = lax.axis_index("s") * _NC + lax.axis_index("c")
        base = wid * (rows_w * D)

        def chunk_body(c, carry):
            off = base + c * words
            pltpu.sync_copy(emb_hbm.at[pl.ds(off, words)], emb_v)
            pltpu.sync_copy(x_hbm.at[b_idx, pl.ds(off, words)], x_v)

            def add_body(i, carry):
                sl = pl.ds(i * _L, _L)
                x_v[sl] = x_v[sl] + emb_v[sl]
                return carry

            lax.fori_loop(0, words // _L, add_body, None)
            pltpu.sync_copy(x_v, out_hbm.at[pl.ds(off, words)])
            return carry

        lax.fori_loop(0, n_chunks, chunk_body, None)

    return k(xf, ef).reshape(1, N, D)


def kernel(x, emb):
    B, N, D = x.shape
    e = emb[:N]
    sc_out = _sc_add_one_batch(x, e, B - 1)
    tc_out = _tc_add(x, e, B - 1)
    return jnp.concatenate([tc_out, sc_out], axis=0)


# speedup vs baseline: 7.7707x; 4.5774x over previous
import jax
import jax.numpy as jnp
from jax.experimental import pallas as pl
from jax.experimental.pallas import tpu as pltpu

_BLK_N = 2048
_BLK_B = 2


def _add_kernel(x_ref, emb_ref, o_ref):
    o_ref[...] = x_ref[...] + emb_ref[...]


def kernel(x, emb):
    B, N, D = x.shape
    nb = N // _BLK_N
    return pl.pallas_call(
        _add_kernel,
        grid=(nb, B // _BLK_B),
        in_specs=[
            pl.BlockSpec((_BLK_B, _BLK_N, D), lambda i, b: (b, i, 0)),
            pl.BlockSpec((_BLK_N, D), lambda i, b: (i, 0)),
        ],
        out_specs=pl.BlockSpec((_BLK_B, _BLK_N, D), lambda i, b: (b, i, 0)),
        out_shape=jax.ShapeDtypeStruct((B, N, D), x.dtype),
        compiler_params=pltpu.CompilerParams(
            dimension_semantics=("parallel", "parallel"),
            vmem_limit_bytes=120 * 1024 * 1024,
        ),
    )(x, emb[:N])
